# baseline (speedup 1.0000x reference)
import functools

import jax
import jax.numpy as jnp
from jax import lax
from jax.experimental import pallas as pl
from jax.experimental.pallas import tpu as pltpu
from jax.experimental.pallas import tpu_sc as plsc

_NC = 1
_NS = 16
_NW = _NC * _NS


def _gather_body(b_per_w, b, table, labels, samples, out, idx_v, rows_v, sem):
    wid = lax.axis_index("s") * _NC + lax.axis_index("c")
    base = wid * b_per_w

    @pl.when(base < b)
    def _():
        pltpu.sync_copy(labels.at[pl.ds(base, b_per_w)], idx_v)

    @pl.when(base >= b)
    def _():
        pltpu.sync_copy(samples.at[pl.ds(base - b, b_per_w)], idx_v)

    pltpu.async_copy(table.at[idx_v], rows_v, sem).wait()
    pltpu.sync_copy(rows_v, out.at[pl.ds(base, b_per_w)])


def _sc_gather(table, labels, samples):
    b, s, d = labels.shape[0], samples.shape[0], table.shape[1]
    b_per_w = (b + s) // _NW
    mesh = plsc.VectorSubcoreMesh(core_axis_name="c", subcore_axis_name="s",
                                  num_cores=_NC)
    return pl.kernel(
        functools.partial(_gather_body, b_per_w, b),
        out_type=jax.ShapeDtypeStruct((b + s, d), table.dtype),
        mesh=mesh,
        scratch_types=[
            pltpu.VMEM((b_per_w,), jnp.int32),
            pltpu.VMEM((b_per_w, d), table.dtype),
            pltpu.SemaphoreType.DMA,
        ],
    )(table, labels, samples)


def _rowsum(m):
    ones = jnp.ones((m.shape[1], 1), jnp.float32)
    return lax.dot_general(m, ones, (((1,), (0,)), ((), ())),
                           preferred_element_type=jnp.float32)


_LOG2E = 1.4426950408889634


def _dense_body(b, x_ref, rows_ref, out_ref):
    x = x_ref[...]
    tw = rows_ref[:b, :]
    sw = rows_ref[b:, :]
    xs = x * _LOG2E
    sws = sw * _LOG2E
    x2 = _rowsum(xs * xs)
    sw2 = _rowsum(sws * sws)
    g2 = lax.dot_general(xs, sws * -2.0, (((1,), (1,)), ((), ())),
                         preferred_element_type=jnp.float32)
    m = jnp.maximum((x2 + g2) + jnp.transpose(sw2), 0.0)
    dist = m * lax.rsqrt(m + 1e-30)
    s = _rowsum(jnp.exp2(dist))
    diff = x - tw
    td2 = _rowsum(diff * diff)
    out_ref[...] = lax.squeeze(jnp.sqrt(td2) - jnp.log(s), (1,))


def _dense(inputs, rows):
    b = inputs.shape[0]
    return pl.pallas_call(
        functools.partial(_dense_body, b),
        out_shape=jax.ShapeDtypeStruct((b,), jnp.float32),
    )(inputs, rows)


def kernel(inputs, labels, sample_ids, weight):
    rows = _sc_gather(weight, labels.astype(jnp.int32),
                      sample_ids.astype(jnp.int32))
    return _dense(inputs, rows)


# speedup vs baseline: 1.0008x; 1.0008x over previous
import functools

import jax
import jax.numpy as jnp
from jax import lax
from jax.experimental import pallas as pl
from jax.experimental.pallas import tpu as pltpu
from jax.experimental.pallas import tpu_sc as plsc

_NC = 1
_NS = 16
_NW = _NC * _NS


def _gather_body(b_per_w, b, table, labels, samples, out, idx_v, rows_v, sem):
    wid = lax.axis_index("s") * _NC + lax.axis_index("c")
    base = wid * b_per_w

    @pl.when(base < b)
    def _():
        pltpu.sync_copy(labels.at[pl.ds(base, b_per_w)], idx_v)

    @pl.when(base >= b)
    def _():
        pltpu.sync_copy(samples.at[pl.ds(base - b, b_per_w)], idx_v)

    pltpu.async_copy(table.at[idx_v], rows_v, sem).wait()
    pltpu.sync_copy(rows_v, out.at[pl.ds(base, b_per_w)])


def _sc_gather(table, labels, samples):
    b, s, d = labels.shape[0], samples.shape[0], table.shape[1]
    b_per_w = (b + s) // _NW
    mesh = plsc.VectorSubcoreMesh(core_axis_name="c", subcore_axis_name="s",
                                  num_cores=_NC)
    return pl.kernel(
        functools.partial(_gather_body, b_per_w, b),
        out_type=jax.ShapeDtypeStruct((b + s, d), table.dtype),
        mesh=mesh,
        scratch_types=[
            pltpu.VMEM((b_per_w,), jnp.int32),
            pltpu.VMEM((b_per_w, d), table.dtype),
            pltpu.SemaphoreType.DMA,
        ],
    )(table, labels, samples)


def _rowsum(m):
    ones = jnp.ones((m.shape[1], 1), jnp.float32)
    return lax.dot_general(m, ones, (((1,), (0,)), ((), ())),
                           preferred_element_type=jnp.float32)


_LOG2E = 1.4426950408889634


def _dense_body(x_ref, blk_ref, out_ref, td2_ref):
    i = pl.program_id(0)
    x = x_ref[...]
    blk = blk_ref[...]

    @pl.when(i == 0)
    def _():
        diff = x - blk
        td2_ref[...] = _rowsum(diff * diff)

    @pl.when(i == 1)
    def _():
        xs = x * _LOG2E
        sws = blk * _LOG2E
        x2 = _rowsum(xs * xs)
        sw2 = _rowsum(sws * sws)
        g2 = lax.dot_general(xs, sws * -2.0, (((1,), (1,)), ((), ())),
                             preferred_element_type=jnp.float32)
        m = jnp.maximum((x2 + g2) + jnp.transpose(sw2), 0.0)
        dist = m * lax.rsqrt(m + 1e-30)
        s = _rowsum(jnp.exp2(dist))
        out_ref[...] = lax.squeeze(jnp.sqrt(td2_ref[...]) - jnp.log(s), (1,))


def _dense(inputs, rows):
    b, d = inputs.shape
    return pl.pallas_call(
        _dense_body,
        grid=(2,),
        in_specs=[pl.BlockSpec((b, d), lambda i: (0, 0)),
                  pl.BlockSpec((b, d), lambda i: (i, 0))],
        out_specs=pl.BlockSpec((b,), lambda i: (0,)),
        out_shape=jax.ShapeDtypeStruct((b,), jnp.float32),
        scratch_shapes=[pltpu.VMEM((b, 1), jnp.float32)],
    )(inputs, rows)


def kernel(inputs, labels, sample_ids, weight):
    rows = _sc_gather(weight, labels.astype(jnp.int32),
                      sample_ids.astype(jnp.int32))
    return _dense(inputs, rows)
